# baseline (speedup 1.0000x reference)
import functools

import jax
import jax.numpy as jnp
from jax import lax
from jax.experimental import pallas as pl
from jax.experimental.pallas import tpu as pltpu
from jax.experimental.pallas import tpu_sc as plsc

N = 10000
N2 = 10240
E = 320000
EN = E + N
NW = 32
S = 128
SUB = 81
T = SUB * S
EP = NW * T
RPT = N2 // 16
C = 128


def _mm_body(x_ref, wl_ref, bl_ref, wr_ref, br_ref, xl_ref, xr_ref):
    h = x_ref[...]
    xl_ref[...] = jnp.dot(h, wl_ref[...], preferred_element_type=jnp.float32) + bl_ref[...]
    xr_ref[...] = jnp.dot(h, wr_ref[...], preferred_element_type=jnp.float32) + br_ref[...]


def _mm(xp, Wl, bl, Wr, br):
    blk = 1024
    return pl.pallas_call(
        _mm_body,
        grid=(N2 // blk,),
        in_specs=[
            pl.BlockSpec((blk, C), lambda i: (i, 0)),
            pl.BlockSpec((C, C), lambda i: (0, 0)),
            pl.BlockSpec((1, C), lambda i: (0, 0)),
            pl.BlockSpec((C, C), lambda i: (0, 0)),
            pl.BlockSpec((1, C), lambda i: (0, 0)),
        ],
        out_specs=[pl.BlockSpec((blk, C), lambda i: (i, 0))] * 2,
        out_shape=[jax.ShapeDtypeStruct((N2, C), jnp.float32)] * 2,
    )(xp, Wl, bl.reshape(1, C), Wr, br.reshape(1, C))


def _cmm_body(o0_ref, o1_ref, r_ref, bias_ref, wl_ref, bl_ref, wr_ref, br_ref,
              xl_ref, xr_ref):
    o = o0_ref[...] + o1_ref[...]
    h = jnp.maximum(o * r_ref[...] + bias_ref[...], 0.0)
    xl_ref[...] = jnp.dot(h, wl_ref[...], preferred_element_type=jnp.float32) + bl_ref[...]
    xr_ref[...] = jnp.dot(h, wr_ref[...], preferred_element_type=jnp.float32) + br_ref[...]


def _cmm(o0, o1, recip, bias, Wl, bl, Wr, br):
    blk = 1024
    return pl.pallas_call(
        _cmm_body,
        grid=(N2 // blk,),
        in_specs=[
            pl.BlockSpec((blk, C), lambda i: (i, 0)),
            pl.BlockSpec((blk, C), lambda i: (i, 0)),
            pl.BlockSpec((blk, 1), lambda i: (i, 0)),
            pl.BlockSpec((1, C), lambda i: (0, 0)),
            pl.BlockSpec((C, C), lambda i: (0, 0)),
            pl.BlockSpec((1, C), lambda i: (0, 0)),
            pl.BlockSpec((C, C), lambda i: (0, 0)),
            pl.BlockSpec((1, C), lambda i: (0, 0)),
        ],
        out_specs=[pl.BlockSpec((blk, C), lambda i: (i, 0))] * 2,
        out_shape=[jax.ShapeDtypeStruct((N2, C), jnp.float32)] * 2,
    )(o0, o1, recip, bias.reshape(1, C), Wl, bl.reshape(1, C), Wr, br.reshape(1, C))


def _final_body(o0_ref, o1_ref, r_ref, bias_ref, y_ref):
    o = o0_ref[...] + o1_ref[...]
    h = jnp.maximum(o * r_ref[...] + bias_ref[...], 0.0)
    m = jnp.max(h, axis=1, keepdims=True)
    s = jnp.sum(jnp.exp(h - m), axis=1, keepdims=True)
    y_ref[...] = h - m - jnp.log(s)


def _final(o0, o1, recip, bias):
    blk = 1024
    return pl.pallas_call(
        _final_body,
        grid=(N2 // blk,),
        in_specs=[
            pl.BlockSpec((blk, C), lambda i: (i, 0)),
            pl.BlockSpec((blk, C), lambda i: (i, 0)),
            pl.BlockSpec((blk, 1), lambda i: (i, 0)),
            pl.BlockSpec((1, C), lambda i: (0, 0)),
        ],
        out_specs=pl.BlockSpec((blk, C), lambda i: (i, 0)),
        out_shape=jax.ShapeDtypeStruct((N2, C), jnp.float32),
    )(o0, o1, recip, bias.reshape(1, C))


_edge_call_cache = []


def _edge_body(xl_hbm, xr_hbm, src_hbm, dst_hbm, att_hbm, z2_hbm, z1_hbm,
               outp_hbm, den_hbm,
               srci, dsti, xlr, xrr, exb, attb, out_sh, den_sh, sem1, sem2):
    cid = lax.axis_index("c")
    sid = lax.axis_index("s")
    wid = cid * 16 + sid
    r0 = sid * RPT

    pltpu.sync_copy(z2_hbm, out_sh.at[pl.ds(r0, RPT)])
    pltpu.sync_copy(z1_hbm, den_sh.at[pl.ds(r0, RPT)])
    pltpu.sync_copy(att_hbm, attb)
    plsc.subcore_barrier()

    base = wid * T
    lanes = lax.iota(jnp.int32, 16)
    ids_g = [g * 16 + lanes for g in range(8)]

    def subchunk(j, carry):
        b = base + j * S
        pltpu.sync_copy(src_hbm.at[pl.ds(b, S)], srci)
        pltpu.sync_copy(dst_hbm.at[pl.ds(b, S)], dsti)
        cp1 = pltpu.async_copy(xl_hbm.at[srci], xlr, sem1)
        cp2 = pltpu.async_copy(xr_hbm.at[dsti], xrr, sem2)
        cp1.wait()
        cp2.wait()

        valids = []
        for g in range(8):
            srcv = plsc.load_gather(srci, [ids_g[g]])
            dstv = plsc.load_gather(dsti, [ids_g[g]])
            idg = (b + g * 16) + lanes
            valids.append(((srcv != dstv) | (idg >= E)) & (idg < EN))

        def cbody(c, accs):
            cs = jnp.zeros((16,), jnp.int32) + c
            attc = plsc.load_gather(attb, [cs])
            out = []
            for g in range(8):
                a = plsc.load_gather(xlr, [ids_g[g], cs])
                bv = plsc.load_gather(xrr, [ids_g[g], cs])
                v = a + bv
                v = jnp.maximum(v, 0.0) + 0.2 * jnp.minimum(v, 0.0)
                out.append(accs[g] + v * attc)
            return tuple(out)

        accs = lax.fori_loop(0, C, cbody,
                             tuple(jnp.zeros((16,), jnp.float32) for _ in range(8)))

        exs = []
        for g in range(8):
            ex = jnp.where(valids[g], jnp.exp(accs[g]), 0.0)
            exs.append(ex)
            exb[pl.ds(g * 16, 16)] = ex

        def mbody(c, carry2):
            cs = jnp.zeros((16,), jnp.int32) + c
            for g in range(8):
                v = plsc.load_gather(xlr, [ids_g[g], cs]) * exs[g]
                plsc.store_scatter(xrr, [ids_g[g], cs], v)
            return carry2

        lax.fori_loop(0, C, mbody, 0)

        pltpu.sync_copy(xrr, out_sh.at[dsti], add=True)
        pltpu.sync_copy(exb, den_sh.at[dsti], add=True)
        return carry

    lax.fori_loop(0, SUB, subchunk, 0)
    plsc.subcore_barrier()

    pltpu.sync_copy(out_sh.at[pl.ds(r0, RPT)], outp_hbm.at[cid, pl.ds(r0, RPT)])
    pltpu.sync_copy(den_sh.at[pl.ds(r0, RPT)], den_hbm.at[cid, pl.ds(r0, RPT)])


def _get_edge_kernel():
    if not _edge_call_cache:
        mesh = plsc.VectorSubcoreMesh(core_axis_name="c", subcore_axis_name="s")
        k = functools.partial(
            pl.kernel,
            mesh=mesh,
            compiler_params=pltpu.CompilerParams(needs_layout_passes=False),
            out_type=(
                jax.ShapeDtypeStruct((2, N2, C), jnp.float32),
                jax.ShapeDtypeStruct((2, N2), jnp.float32),
            ),
            scratch_types=[
                pltpu.VMEM((S,), jnp.int32),
                pltpu.VMEM((S,), jnp.int32),
                pltpu.VMEM((S, C), jnp.float32),
                pltpu.VMEM((S, C), jnp.float32),
                pltpu.VMEM((S,), jnp.float32),
                pltpu.VMEM((C,), jnp.float32),
                pltpu.VMEM_SHARED((N2, C), jnp.float32),
                pltpu.VMEM_SHARED((N2,), jnp.float32),
                pltpu.SemaphoreType.DMA,
                pltpu.SemaphoreType.DMA,
            ],
        )(_edge_body)
        _edge_call_cache.append(k)
    return _edge_call_cache[0]


def kernel(x, edge_index, Wl0, bl0, Wr0, br0, att0, bias0, Wl1, bl1, Wr1, br1,
           att1, bias1, Wl2, bl2, Wr2, br2, att2, bias2):
    xp = jnp.pad(x, ((0, N2 - N), (0, 0)))
    loop = jnp.arange(N, dtype=jnp.int32)
    pad = jnp.zeros((EP - EN,), jnp.int32)
    srcp = jnp.concatenate([edge_index[0], loop, pad])
    dstp = jnp.concatenate([edge_index[1], loop, pad])
    z2 = jnp.zeros((RPT, C), jnp.float32)
    z1 = jnp.zeros((RPT,), jnp.float32)
    edge = _get_edge_kernel()

    def sc_layer(xl, xr, att):
        outp, den = edge(xl, xr, srcp, dstp, att.reshape(C), z2, z1)
        recip = (1.0 / (den[0] + den[1] + 1e-16)).reshape(N2, 1)
        return outp[0], outp[1], recip

    xl, xr = _mm(xp, Wl0, bl0, Wr0, br0)
    o0, o1, r = sc_layer(xl, xr, att0)
    xl, xr = _cmm(o0, o1, r, bias0, Wl1, bl1, Wr1, br1)
    o0, o1, r = sc_layer(xl, xr, att1)
    xl, xr = _cmm(o0, o1, r, bias1, Wl2, bl2, Wr2, br2)
    o0, o1, r = sc_layer(xl, xr, att2)
    y = _final(o0, o1, r, bias2)
    return y[:N]


# speedup vs baseline: 1.6491x; 1.6491x over previous
import functools

import jax
import jax.numpy as jnp
from jax import lax
from jax.experimental import pallas as pl
from jax.experimental.pallas import tpu as pltpu
from jax.experimental.pallas import tpu_sc as plsc

N = 10000
N2 = 10240
E = 320000
EN = E + N
NW = 32
S = 64
SUB = 164
T = SUB * S
EP = NW * T
RPT = N2 // 16
G = S // 16
C = 128


def _mm_body(x_ref, wl_ref, bl_ref, wr_ref, br_ref, xl_ref, xr_ref):
    h = x_ref[...]
    xl_ref[...] = jnp.dot(h, wl_ref[...], preferred_element_type=jnp.float32) + bl_ref[...]
    xr_ref[...] = jnp.dot(h, wr_ref[...], preferred_element_type=jnp.float32) + br_ref[...]


def _mm(xp, Wl, bl, Wr, br):
    blk = 1024
    return pl.pallas_call(
        _mm_body,
        grid=(N2 // blk,),
        in_specs=[
            pl.BlockSpec((blk, C), lambda i: (i, 0)),
            pl.BlockSpec((C, C), lambda i: (0, 0)),
            pl.BlockSpec((1, C), lambda i: (0, 0)),
            pl.BlockSpec((C, C), lambda i: (0, 0)),
            pl.BlockSpec((1, C), lambda i: (0, 0)),
        ],
        out_specs=[pl.BlockSpec((blk, C), lambda i: (i, 0))] * 2,
        out_shape=[jax.ShapeDtypeStruct((N2, C), jnp.float32)] * 2,
    )(xp, Wl, bl.reshape(1, C), Wr, br.reshape(1, C))


def _cmm_body(o0_ref, o1_ref, r_ref, bias_ref, wl_ref, bl_ref, wr_ref, br_ref,
              xl_ref, xr_ref):
    o = o0_ref[...] + o1_ref[...]
    h = jnp.maximum(o * r_ref[...] + bias_ref[...], 0.0)
    xl_ref[...] = jnp.dot(h, wl_ref[...], preferred_element_type=jnp.float32) + bl_ref[...]
    xr_ref[...] = jnp.dot(h, wr_ref[...], preferred_element_type=jnp.float32) + br_ref[...]


def _cmm(o0, o1, recip, bias, Wl, bl, Wr, br):
    blk = 1024
    return pl.pallas_call(
        _cmm_body,
        grid=(N2 // blk,),
        in_specs=[
            pl.BlockSpec((blk, C), lambda i: (i, 0)),
            pl.BlockSpec((blk, C), lambda i: (i, 0)),
            pl.BlockSpec((blk, 1), lambda i: (i, 0)),
            pl.BlockSpec((1, C), lambda i: (0, 0)),
            pl.BlockSpec((C, C), lambda i: (0, 0)),
            pl.BlockSpec((1, C), lambda i: (0, 0)),
            pl.BlockSpec((C, C), lambda i: (0, 0)),
            pl.BlockSpec((1, C), lambda i: (0, 0)),
        ],
        out_specs=[pl.BlockSpec((blk, C), lambda i: (i, 0))] * 2,
        out_shape=[jax.ShapeDtypeStruct((N2, C), jnp.float32)] * 2,
    )(o0, o1, recip, bias.reshape(1, C), Wl, bl.reshape(1, C), Wr, br.reshape(1, C))


def _final_body(o0_ref, o1_ref, r_ref, bias_ref, y_ref):
    o = o0_ref[...] + o1_ref[...]
    h = jnp.maximum(o * r_ref[...] + bias_ref[...], 0.0)
    m = jnp.max(h, axis=1, keepdims=True)
    s = jnp.sum(jnp.exp(h - m), axis=1, keepdims=True)
    y_ref[...] = h - m - jnp.log(s)


def _final(o0, o1, recip, bias):
    blk = 1024
    return pl.pallas_call(
        _final_body,
        grid=(N2 // blk,),
        in_specs=[
            pl.BlockSpec((blk, C), lambda i: (i, 0)),
            pl.BlockSpec((blk, C), lambda i: (i, 0)),
            pl.BlockSpec((blk, 1), lambda i: (i, 0)),
            pl.BlockSpec((1, C), lambda i: (0, 0)),
        ],
        out_specs=pl.BlockSpec((blk, C), lambda i: (i, 0)),
        out_shape=jax.ShapeDtypeStruct((N2, C), jnp.float32),
    )(o0, o1, recip, bias.reshape(1, C))


_edge_call_cache = []


def _edge_body(xl_hbm, xr_hbm, src_hbm, dst_hbm, att_hbm, z2_hbm, z1_hbm,
               outp_hbm, den_hbm,
               srci0, srci1, dsti0, dsti1, dstS0, dstS1,
               xlr0, xlr1, xrr0, xrr1, exb0, exb1, attb,
               out_sh, den_sh,
               semI0, semI1, semJ0, semJ1, semL0, semL1, semR0, semR1,
               ssem0, ssem1, dsem0, dsem1):
    cid = lax.axis_index("c")
    sid = lax.axis_index("s")
    wid = cid * 16 + sid
    r0 = sid * RPT
    srci = (srci0, srci1)
    dsti = (dsti0, dsti1)
    dstS = (dstS0, dstS1)
    xlr = (xlr0, xlr1)
    xrr = (xrr0, xrr1)
    exb = (exb0, exb1)
    semI = (semI0, semI1)
    semJ = (semJ0, semJ1)
    semL = (semL0, semL1)
    semR = (semR0, semR1)
    ssem = (ssem0, ssem1)
    dsem = (dsem0, dsem1)

    pltpu.sync_copy(z2_hbm, out_sh.at[pl.ds(r0, RPT)])
    pltpu.sync_copy(z1_hbm, den_sh.at[pl.ds(r0, RPT)])
    pltpu.sync_copy(att_hbm, attb)
    plsc.subcore_barrier()

    lanes = lax.iota(jnp.int32, 16)
    ids_g = [g * 16 + lanes for g in range(G)]

    def start_idx(j, b):
        pltpu.async_copy(src_hbm.at[wid, j], srci[b], semI[b])
        pltpu.async_copy(dst_hbm.at[wid, j], dsti[b], semJ[b])

    def wait_idx(j, b):
        pltpu.make_async_copy(src_hbm.at[wid, j], srci[b], semI[b]).wait()
        pltpu.make_async_copy(dst_hbm.at[wid, j], dsti[b], semJ[b]).wait()

    def start_gather(j, b):
        pltpu.async_copy(xl_hbm.at[srci[b]], xlr[b], semL[b])
        pltpu.async_copy(xr_hbm.at[dsti[b]], xrr[b], semR[b])

    def wait_gather(j, b):
        pltpu.make_async_copy(xl_hbm.at[srci[b]], xlr[b], semL[b]).wait()
        pltpu.make_async_copy(xr_hbm.at[dsti[b]], xrr[b], semR[b]).wait()

    def wait_out_scatter(j, b):
        pltpu.make_async_copy(xrr[b], out_sh.at[dstS[b]], ssem[b]).wait()

    def wait_den_scatter(j, b):
        pltpu.make_async_copy(exb[b], den_sh.at[dstS[b]], dsem[b]).wait()

    start_idx(0, 0)
    start_idx(1, 1)
    wait_idx(0, 0)
    start_gather(0, 0)

    def do_subchunk(j, b):
        b1 = 1 - b
        wait_gather(j, b)

        @pl.when(j >= 2)
        def _():
            wait_den_scatter(j - 2, b)

        base = wid * T + j * S
        valids = []
        for g in range(G):
            srcv = plsc.load_gather(srci[b], [ids_g[g]])
            dstv = plsc.load_gather(dsti[b], [ids_g[g]])
            dstS[b][pl.ds(g * 16, 16)] = dstv
            idg = (base + g * 16) + lanes
            valids.append(((srcv != dstv) | (idg >= E)) & (idg < EN))

        @pl.when(j + 2 < SUB)
        def _():
            start_idx(j + 2, b)

        def cbody(c, accs):
            cs = jnp.zeros((16,), jnp.int32) + c
            attc = plsc.load_gather(attb, [cs])
            out = []
            for g in range(G):
                a = plsc.load_gather(xlr[b], [ids_g[g], cs])
                bv = plsc.load_gather(xrr[b], [ids_g[g], cs])
                v = a + bv
                v = jnp.maximum(v, 0.0) + 0.2 * jnp.minimum(v, 0.0)
                out.append(accs[g] + v * attc)
            return tuple(out)

        accs = lax.fori_loop(0, C, cbody,
                             tuple(jnp.zeros((16,), jnp.float32) for _ in range(G)))

        for g in range(G):
            ex = jnp.where(valids[g], jnp.exp(accs[g]), 0.0)
            exb[b][pl.ds(g * 16, 16)] = ex

        @pl.when((j >= 1) & (j + 1 < SUB))
        def _():
            wait_out_scatter(j - 1, b1)

        @pl.when(j + 1 < SUB)
        def _():
            wait_idx(j + 1, b1)
            start_gather(j + 1, b1)

        def mrow(e):
            exv = plsc.load_gather(exb[b], [jnp.zeros((16,), jnp.int32) + e])
            for k in range(8):
                sl = pl.ds(k * 16, 16)
                xrr[b][e, sl] = xlr[b][e, sl] * exv

        def mbody(i, carry2):
            mrow(2 * i)
            mrow(2 * i + 1)
            return carry2

        lax.fori_loop(0, S // 2, mbody, 0)

        pltpu.async_copy(xrr[b], out_sh.at[dstS[b]], ssem[b], add=True)
        pltpu.async_copy(exb[b], den_sh.at[dstS[b]], dsem[b], add=True)

    def pair(jp, carry):
        do_subchunk(jp * 2, 0)
        do_subchunk(jp * 2 + 1, 1)
        return carry

    lax.fori_loop(0, SUB // 2, pair, 0)
    wait_out_scatter(SUB - 2, 0)
    wait_den_scatter(SUB - 2, 0)
    wait_out_scatter(SUB - 1, 1)
    wait_den_scatter(SUB - 1, 1)
    plsc.subcore_barrier()

    pltpu.sync_copy(out_sh.at[pl.ds(r0, RPT)], outp_hbm.at[cid, pl.ds(r0, RPT)])
    pltpu.sync_copy(den_sh.at[pl.ds(r0, RPT)], den_hbm.at[cid, pl.ds(r0, RPT)])


def _get_edge_kernel():
    if not _edge_call_cache:
        mesh = plsc.VectorSubcoreMesh(core_axis_name="c", subcore_axis_name="s")
        k = functools.partial(
            pl.kernel,
            mesh=mesh,
            compiler_params=pltpu.CompilerParams(needs_layout_passes=False),
            out_type=(
                jax.ShapeDtypeStruct((2, N2, C), jnp.float32),
                jax.ShapeDtypeStruct((2, N2), jnp.float32),
            ),
            scratch_types=(
                [pltpu.VMEM((S,), jnp.int32)] * 6
                + [pltpu.VMEM((S, C), jnp.float32)] * 4
                + [pltpu.VMEM((S,), jnp.float32)] * 2
                + [
                    pltpu.VMEM((C,), jnp.float32),
                    pltpu.VMEM_SHARED((N2, C), jnp.float32),
                    pltpu.VMEM_SHARED((N2,), jnp.float32),
                ]
                + [pltpu.SemaphoreType.DMA] * 12
            ),
        )(_edge_body)
        _edge_call_cache.append(k)
    return _edge_call_cache[0]


def kernel(x, edge_index, Wl0, bl0, Wr0, br0, att0, bias0, Wl1, bl1, Wr1, br1,
           att1, bias1, Wl2, bl2, Wr2, br2, att2, bias2):
    xp = jnp.pad(x, ((0, N2 - N), (0, 0)))
    loop = jnp.arange(N, dtype=jnp.int32)
    pad = jnp.zeros((EP - EN,), jnp.int32)
    srcp = jnp.concatenate([edge_index[0], loop, pad]).reshape(NW, SUB, S)
    dstp = jnp.concatenate([edge_index[1], loop, pad]).reshape(NW, SUB, S)
    z2 = jnp.zeros((RPT, C), jnp.float32)
    z1 = jnp.zeros((RPT,), jnp.float32)
    edge = _get_edge_kernel()

    def sc_layer(xl, xr, att):
        outp, den = edge(xl, xr, srcp, dstp, att.reshape(C), z2, z1)
        recip = (1.0 / (den[0] + den[1] + 1e-16)).reshape(N2, 1)
        return outp[0], outp[1], recip

    xl, xr = _mm(xp, Wl0, bl0, Wr0, br0)
    o0, o1, r = sc_layer(xl, xr, att0)
    xl, xr = _cmm(o0, o1, r, bias0, Wl1, bl1, Wr1, br1)
    o0, o1, r = sc_layer(xl, xr, att1)
    xl, xr = _cmm(o0, o1, r, bias1, Wl2, bl2, Wr2, br2)
    o0, o1, r = sc_layer(xl, xr, att2)
    y = _final(o0, o1, r, bias2)
    return y[:N]


# speedup vs baseline: 1.7852x; 1.0825x over previous
import functools

import jax
import jax.numpy as jnp
from jax import lax
from jax.experimental import pallas as pl
from jax.experimental.pallas import tpu as pltpu
from jax.experimental.pallas import tpu_sc as plsc

N = 10000
N2 = 10240
E = 320000
EN = E + N
NW = 32
S = 48
SUB = 220
T = SUB * S
EP = NW * T
RPT = N2 // 16
G = S // 16
C = 128


def _mm_body(x_ref, wl_ref, bl_ref, wr_ref, br_ref, xl_ref, xr_ref):
    h = x_ref[...]
    xl_ref[...] = jnp.dot(h, wl_ref[...], preferred_element_type=jnp.float32) + bl_ref[...]
    xr_ref[...] = jnp.dot(h, wr_ref[...], preferred_element_type=jnp.float32) + br_ref[...]


def _mm(xp, Wl, bl, Wr, br):
    blk = 1024
    return pl.pallas_call(
        _mm_body,
        grid=(N2 // blk,),
        in_specs=[
            pl.BlockSpec((blk, C), lambda i: (i, 0)),
            pl.BlockSpec((C, C), lambda i: (0, 0)),
            pl.BlockSpec((1, C), lambda i: (0, 0)),
            pl.BlockSpec((C, C), lambda i: (0, 0)),
            pl.BlockSpec((1, C), lambda i: (0, 0)),
        ],
        out_specs=[pl.BlockSpec((blk, C), lambda i: (i, 0))] * 2,
        out_shape=[jax.ShapeDtypeStruct((N2, C), jnp.float32)] * 2,
    )(xp, Wl, bl.reshape(1, C), Wr, br.reshape(1, C))


def _cmm_body(o0_ref, o1_ref, r_ref, bias_ref, wl_ref, bl_ref, wr_ref, br_ref,
              xl_ref, xr_ref):
    o = o0_ref[...] + o1_ref[...]
    h = jnp.maximum(o * r_ref[...] + bias_ref[...], 0.0)
    xl_ref[...] = jnp.dot(h, wl_ref[...], preferred_element_type=jnp.float32) + bl_ref[...]
    xr_ref[...] = jnp.dot(h, wr_ref[...], preferred_element_type=jnp.float32) + br_ref[...]


def _cmm(o0, o1, recip, bias, Wl, bl, Wr, br):
    blk = 1024
    return pl.pallas_call(
        _cmm_body,
        grid=(N2 // blk,),
        in_specs=[
            pl.BlockSpec((blk, C), lambda i: (i, 0)),
            pl.BlockSpec((blk, C), lambda i: (i, 0)),
            pl.BlockSpec((blk, 1), lambda i: (i, 0)),
            pl.BlockSpec((1, C), lambda i: (0, 0)),
            pl.BlockSpec((C, C), lambda i: (0, 0)),
            pl.BlockSpec((1, C), lambda i: (0, 0)),
            pl.BlockSpec((C, C), lambda i: (0, 0)),
            pl.BlockSpec((1, C), lambda i: (0, 0)),
        ],
        out_specs=[pl.BlockSpec((blk, C), lambda i: (i, 0))] * 2,
        out_shape=[jax.ShapeDtypeStruct((N2, C), jnp.float32)] * 2,
    )(o0, o1, recip, bias.reshape(1, C), Wl, bl.reshape(1, C), Wr, br.reshape(1, C))


def _final_body(o0_ref, o1_ref, r_ref, bias_ref, y_ref):
    o = o0_ref[...] + o1_ref[...]
    h = jnp.maximum(o * r_ref[...] + bias_ref[...], 0.0)
    m = jnp.max(h, axis=1, keepdims=True)
    s = jnp.sum(jnp.exp(h - m), axis=1, keepdims=True)
    y_ref[...] = h - m - jnp.log(s)


def _final(o0, o1, recip, bias):
    blk = 1024
    return pl.pallas_call(
        _final_body,
        grid=(N2 // blk,),
        in_specs=[
            pl.BlockSpec((blk, C), lambda i: (i, 0)),
            pl.BlockSpec((blk, C), lambda i: (i, 0)),
            pl.BlockSpec((blk, 1), lambda i: (i, 0)),
            pl.BlockSpec((1, C), lambda i: (0, 0)),
        ],
        out_specs=pl.BlockSpec((blk, C), lambda i: (i, 0)),
        out_shape=jax.ShapeDtypeStruct((N2, C), jnp.float32),
    )(o0, o1, recip, bias.reshape(1, C))


_edge_call_cache = []


def _edge_body(xl_hbm, xr_hbm, idx_hbm, att_hbm, z2_hbm, z1_hbm,
               outp_hbm, den_hbm,
               idx0, idx1, dstS0, dstS1,
               xlr0, xlr1, xrr0, xrr1, msg0, msg1, exb0, exb1, attb,
               out_sh, den_sh,
               semI0, semI1, semLa0, semLa1, semLb0, semLb1,
               semRa0, semRa1, semRb0, semRb1,
               ssem0, ssem1, dsem0, dsem1):
    cid = lax.axis_index("c")
    sid = lax.axis_index("s")
    wid = cid * 16 + sid
    r0 = sid * RPT
    idxb = (idx0, idx1)
    dstS = (dstS0, dstS1)
    xlr = (xlr0, xlr1)
    xrr = (xrr0, xrr1)
    msg = (msg0, msg1)
    exb = (exb0, exb1)
    semI = (semI0, semI1)
    semLa = (semLa0, semLa1)
    semLb = (semLb0, semLb1)
    semRa = (semRa0, semRa1)
    semRb = (semRb0, semRb1)
    ssem = (ssem0, ssem1)
    dsem = (dsem0, dsem1)
    H = S // 2

    pltpu.sync_copy(z2_hbm, out_sh.at[pl.ds(r0, RPT)])
    pltpu.sync_copy(z1_hbm, den_sh.at[pl.ds(r0, RPT)])
    pltpu.sync_copy(att_hbm, attb)
    plsc.subcore_barrier()

    lanes = lax.iota(jnp.int32, 16)
    ids_g = [g * 16 + lanes for g in range(G)]

    def start_idx(j, b):
        pltpu.async_copy(idx_hbm.at[wid, j], idxb[b], semI[b])

    def wait_idx(j, b):
        pltpu.make_async_copy(idx_hbm.at[wid, j], idxb[b], semI[b]).wait()

    def gather_descs(b):
        return (
            pltpu.make_async_copy(
                xl_hbm.at[idxb[b].at[0, pl.ds(0, H)]], xlr[b].at[pl.ds(0, H)], semLa[b]),
            pltpu.make_async_copy(
                xl_hbm.at[idxb[b].at[0, pl.ds(H, H)]], xlr[b].at[pl.ds(H, H)], semLb[b]),
            pltpu.make_async_copy(
                xr_hbm.at[idxb[b].at[1, pl.ds(0, H)]], xrr[b].at[pl.ds(0, H)], semRa[b]),
            pltpu.make_async_copy(
                xr_hbm.at[idxb[b].at[1, pl.ds(H, H)]], xrr[b].at[pl.ds(H, H)], semRb[b]),
        )

    def start_gather(j, b):
        for d in gather_descs(b):
            d.start()

    def wait_gather(j, b):
        for d in gather_descs(b):
            d.wait()

    def wait_out_scatter(j, b):
        pltpu.make_async_copy(msg[b], out_sh.at[dstS[b]], ssem[b]).wait()

    def wait_den_scatter(j, b):
        pltpu.make_async_copy(exb[b], den_sh.at[dstS[b]], dsem[b]).wait()

    start_idx(0, 0)
    start_idx(1, 1)
    wait_idx(0, 0)
    start_gather(0, 0)

    def do_subchunk(j, b):
        b1 = 1 - b
        wait_gather(j, b)

        @pl.when(j + 1 < SUB)
        def _():
            wait_idx(j + 1, b1)
            start_gather(j + 1, b1)

        @pl.when(j >= 2)
        def _():
            wait_out_scatter(j - 2, b)
            wait_den_scatter(j - 2, b)

        base = wid * T + j * S
        zero16 = jnp.zeros((16,), jnp.int32)
        valids = []
        for g in range(G):
            srcv = plsc.load_gather(idxb[b], [zero16, ids_g[g]])
            dstv = plsc.load_gather(idxb[b], [zero16 + 1, ids_g[g]])
            dstS[b][pl.ds(g * 16, 16)] = dstv
            idg = (base + g * 16) + lanes
            valids.append(((srcv != dstv) | (idg >= E)) & (idg < EN))

        @pl.when(j + 2 < SUB)
        def _():
            start_idx(j + 2, b)

        def cbody(c, accs):
            cs = jnp.zeros((16,), jnp.int32) + c
            attc = plsc.load_gather(attb, [cs])
            out = []
            for g in range(G):
                a = plsc.load_gather(xlr[b], [ids_g[g], cs])
                bv = plsc.load_gather(xrr[b], [ids_g[g], cs])
                v = a + bv
                v = jnp.maximum(v, 0.0) + 0.2 * jnp.minimum(v, 0.0)
                out.append(accs[g] + v * attc)
            return tuple(out)

        accs = lax.fori_loop(0, C, cbody,
                             tuple(jnp.zeros((16,), jnp.float32) for _ in range(G)))

        for g in range(G):
            ex = jnp.where(valids[g], jnp.exp(accs[g]), 0.0)
            exb[b][pl.ds(g * 16, 16)] = ex

        def mrow(e):
            exv = plsc.load_gather(exb[b], [jnp.zeros((16,), jnp.int32) + e])
            for k in range(8):
                sl = pl.ds(k * 16, 16)
                msg[b][e, sl] = xlr[b][e, sl] * exv

        def mbody(i, carry2):
            mrow(2 * i)
            mrow(2 * i + 1)
            return carry2

        lax.fori_loop(0, S // 2, mbody, 0)

        pltpu.async_copy(msg[b], out_sh.at[dstS[b]], ssem[b], add=True)
        pltpu.async_copy(exb[b], den_sh.at[dstS[b]], dsem[b], add=True)

    def pair(jp, carry):
        do_subchunk(jp * 2, 0)
        do_subchunk(jp * 2 + 1, 1)
        return carry

    lax.fori_loop(0, SUB // 2, pair, 0)
    wait_out_scatter(SUB - 2, 0)
    wait_den_scatter(SUB - 2, 0)
    wait_out_scatter(SUB - 1, 1)
    wait_den_scatter(SUB - 1, 1)
    plsc.subcore_barrier()

    pltpu.sync_copy(out_sh.at[pl.ds(r0, RPT)], outp_hbm.at[cid, pl.ds(r0, RPT)])
    pltpu.sync_copy(den_sh.at[pl.ds(r0, RPT)], den_hbm.at[cid, pl.ds(r0, RPT)])


def _get_edge_kernel():
    if not _edge_call_cache:
        mesh = plsc.VectorSubcoreMesh(core_axis_name="c", subcore_axis_name="s")
        k = functools.partial(
            pl.kernel,
            mesh=mesh,
            compiler_params=pltpu.CompilerParams(needs_layout_passes=False),
            out_type=(
                jax.ShapeDtypeStruct((2, N2, C), jnp.float32),
                jax.ShapeDtypeStruct((2, N2), jnp.float32),
            ),
            scratch_types=(
                [pltpu.VMEM((2, S), jnp.int32)] * 2
                + [pltpu.VMEM((S,), jnp.int32)] * 2
                + [pltpu.VMEM((S, C), jnp.float32)] * 6
                + [pltpu.VMEM((S,), jnp.float32)] * 2
                + [
                    pltpu.VMEM((C,), jnp.float32),
                    pltpu.VMEM_SHARED((N2, C), jnp.float32),
                    pltpu.VMEM_SHARED((N2,), jnp.float32),
                ]
                + [pltpu.SemaphoreType.DMA] * 14
            ),
        )(_edge_body)
        _edge_call_cache.append(k)
    return _edge_call_cache[0]


def kernel(x, edge_index, Wl0, bl0, Wr0, br0, att0, bias0, Wl1, bl1, Wr1, br1,
           att1, bias1, Wl2, bl2, Wr2, br2, att2, bias2):
    xp = jnp.pad(x, ((0, N2 - N), (0, 0)))
    loop = jnp.arange(N, dtype=jnp.int32)
    pad = jnp.zeros((EP - EN,), jnp.int32)
    srcp = jnp.concatenate([edge_index[0], loop, pad]).reshape(NW, SUB, 1, S)
    dstp = jnp.concatenate([edge_index[1], loop, pad]).reshape(NW, SUB, 1, S)
    idxp = jnp.concatenate([srcp, dstp], axis=2)
    z2 = jnp.zeros((RPT, C), jnp.float32)
    z1 = jnp.zeros((RPT,), jnp.float32)
    edge = _get_edge_kernel()

    def sc_layer(xl, xr, att):
        outp, den = edge(xl, xr, idxp, att.reshape(C), z2, z1)
        recip = (1.0 / (den[0] + den[1] + 1e-16)).reshape(N2, 1)
        return outp[0], outp[1], recip

    xl, xr = _mm(xp, Wl0, bl0, Wr0, br0)
    o0, o1, r = sc_layer(xl, xr, att0)
    xl, xr = _cmm(o0, o1, r, bias0, Wl1, bl1, Wr1, br1)
    o0, o1, r = sc_layer(xl, xr, att1)
    xl, xr = _cmm(o0, o1, r, bias1, Wl2, bl2, Wr2, br2)
    o0, o1, r = sc_layer(xl, xr, att2)
    y = _final(o0, o1, r, bias2)
    return y[:N]
